# baseline (speedup 1.0000x reference)
import functools

import jax
import jax.numpy as jnp
from jax import lax
from jax.experimental import pallas as pl
from jax.experimental.pallas import tpu as pltpu
from jax.experimental.pallas import tpu_sc as plsc

_B = 16384
_F = 26
_V = 100000
_D = 50
_DP = 128
_NUM = 13
_BF = _B * _F

_NC = 2
_NS = 16
_NW = _NC * _NS

_ROWS_PER_W = _BF // _NW
_GROUP = 128
_GROUPS_PER_CHUNK = 8
_CHUNK = _GROUP * _GROUPS_PER_CHUNK
_CHUNKS_PER_W = _ROWS_PER_W // _CHUNK

_PAD_BLK = 8000


def _pad_body(t_ref, out_ref):
    x = t_ref[...].astype(jnp.bfloat16)
    z = jnp.zeros((_PAD_BLK, _DP - _D), dtype=jnp.bfloat16)
    out_ref[...] = jnp.concatenate([x, z], axis=1)


def _tc_pad(table2d):
    return pl.pallas_call(
        _pad_body,
        grid=(_F * _V // _PAD_BLK,),
        in_specs=[pl.BlockSpec((_PAD_BLK, _D), lambda i: (i, 0))],
        out_specs=pl.BlockSpec((_PAD_BLK, _DP), lambda i: (i, 0)),
        out_shape=jax.ShapeDtypeStruct((_F * _V, _DP), jnp.bfloat16),
    )(table2d)


def _sc_gather_body(table_hbm, idx_hbm, out_hbm, idx_v, rows_v, sem):
    wid = lax.axis_index("s") * _NC + lax.axis_index("c")
    group_base = wid * (_ROWS_PER_W // _GROUP)

    def chunk_step(c, carry):
        g0 = group_base + c * _GROUPS_PER_CHUNK
        pltpu.sync_copy(idx_hbm.at[pl.ds(g0, _GROUPS_PER_CHUNK)], idx_v)
        copies = []
        for j in range(_GROUPS_PER_CHUNK):
            copies.append(
                pltpu.async_copy(
                    table_hbm.at[idx_v.at[j]],
                    rows_v.at[pl.ds(j * _GROUP, _GROUP)],
                    sem,
                )
            )
        for cp in copies:
            cp.wait()
        pltpu.sync_copy(rows_v, out_hbm.at[pl.ds(g0 * _GROUP, _CHUNK)])
        return carry

    lax.fori_loop(0, _CHUNKS_PER_W, chunk_step, 0)


@functools.lru_cache(maxsize=None)
def _make_sc_gather():
    @functools.partial(
        pl.kernel,
        out_type=jax.ShapeDtypeStruct((_BF, _DP), jnp.bfloat16),
        mesh=plsc.VectorSubcoreMesh(core_axis_name="c", subcore_axis_name="s"),
        scratch_types=[
            pltpu.VMEM((_GROUPS_PER_CHUNK, _GROUP), jnp.int32),
            pltpu.VMEM((_CHUNK, _DP), jnp.bfloat16),
            pltpu.SemaphoreType.DMA,
        ],
        compiler_params=pltpu.CompilerParams(use_tc_tiling_on_sc=False),
    )
    def _sc_gather(table_hbm, idx_hbm, out_hbm, idx_v, rows_v, sem):
        _sc_gather_body(table_hbm, idx_hbm, out_hbm, idx_v, rows_v, sem)

    return _sc_gather


_MLP_BLK = 1024
_EMBW = _F * _DP


def _mlp_body(xn_ref, emb_ref, w1a_ref, w1b_ref, b1_ref, w2_ref, b2_ref,
              w3_ref, b3_ref, out_ref):
    h = jnp.dot(xn_ref[...], w1a_ref[...], preferred_element_type=jnp.float32)
    h = h + jnp.dot(emb_ref[...], w1b_ref[...],
                    preferred_element_type=jnp.float32)
    h = jnp.maximum(h + b1_ref[...], 0.0)
    h = jnp.maximum(
        jnp.dot(h, w2_ref[...], preferred_element_type=jnp.float32)
        + b2_ref[...], 0.0)
    out_ref[...] = (
        jnp.dot(h, w3_ref[...], preferred_element_type=jnp.float32)
        + b3_ref[...])


def _tc_mlp(x_num, emb, W1a, W1b, b1, W2, b2, W3, b3):
    grid = (_B // _MLP_BLK,)
    return pl.pallas_call(
        _mlp_body,
        grid=grid,
        in_specs=[
            pl.BlockSpec((_MLP_BLK, _NUM), lambda i: (i, 0)),
            pl.BlockSpec((_MLP_BLK, _EMBW), lambda i: (i, 0)),
            pl.BlockSpec((_NUM, 512), lambda i: (0, 0)),
            pl.BlockSpec((_EMBW, 512), lambda i: (0, 0)),
            pl.BlockSpec((1, 512), lambda i: (0, 0)),
            pl.BlockSpec((512, 256), lambda i: (0, 0)),
            pl.BlockSpec((1, 256), lambda i: (0, 0)),
            pl.BlockSpec((256, 1), lambda i: (0, 0)),
            pl.BlockSpec((1, 1), lambda i: (0, 0)),
        ],
        out_specs=pl.BlockSpec((_MLP_BLK, 1), lambda i: (i, 0)),
        out_shape=jax.ShapeDtypeStruct((_B, 1), jnp.float32),
    )(x_num, emb, W1a, W1b, b1, W2, b2, W3, b3)


def kernel(x_num, x_cat, tables, W1, b1, W2, b2, W3, b3):
    table_pad = _tc_pad(tables.reshape(_F * _V, _D))
    idx = (x_cat.astype(jnp.int32)
           + jnp.arange(_F, dtype=jnp.int32)[None, :] * _V)
    idx2 = idx.reshape(_BF // _GROUP, _GROUP)
    emb = _make_sc_gather()(table_pad, idx2)
    emb2 = emb.reshape(_B, _EMBW)
    w1e = W1[_NUM:].reshape(_F, _D, 512)
    w1e = jnp.pad(w1e, ((0, 0), (0, _DP - _D), (0, 0)))
    w1e = w1e.reshape(_EMBW, 512).astype(jnp.bfloat16)
    out = _tc_mlp(
        x_num, emb2,
        W1[:_NUM], w1e,
        b1.reshape(1, 512), W2, b2.reshape(1, 256), W3, b3.reshape(1, 1))
    return out


# speedup vs baseline: 1.9570x; 1.9570x over previous
import functools

import jax
import jax.numpy as jnp
from jax import lax
from jax.experimental import pallas as pl
from jax.experimental.pallas import tpu as pltpu
from jax.experimental.pallas import tpu_sc as plsc

_B = 16384
_F = 26
_V = 100000
_D = 50
_DP = 128
_NUM = 13
_BF = _B * _F

_NC = 2
_NS = 16
_NW = _NC * _NS

_ROWS_PER_W = _BF // _NW
_GROUP = 128
_GROUPS_PER_CHUNK = 4
_CHUNK = _GROUP * _GROUPS_PER_CHUNK
_CHUNKS_PER_W = _ROWS_PER_W // _CHUNK

_PAD_BLK = 10000


def _pad_body(t_ref, out_ref):
    x = t_ref[0]
    z = jnp.zeros((_PAD_BLK, _DP - _D), dtype=jnp.float32)
    out_ref[...] = jnp.concatenate([x, z], axis=1)


def _tc_pad(tables3d):
    nv = _V // _PAD_BLK
    return pl.pallas_call(
        _pad_body,
        grid=(_F, nv),
        in_specs=[pl.BlockSpec((1, _PAD_BLK, _D), lambda f, j: (f, j, 0))],
        out_specs=pl.BlockSpec((_PAD_BLK, _DP),
                               lambda f, j: (f * (_V // _PAD_BLK) + j, 0)),
        out_shape=jax.ShapeDtypeStruct((_F * _V, _DP), jnp.float32),
    )(tables3d)


def _sc_gather_body(table_hbm, idx_hbm, out_hbm, idx_v, rows_v, sem):
    wid = lax.axis_index("s") * _NC + lax.axis_index("c")
    group_base = wid * (_ROWS_PER_W // _GROUP)

    def chunk_step(c, carry):
        g0 = group_base + c * _GROUPS_PER_CHUNK
        pltpu.sync_copy(idx_hbm.at[pl.ds(g0, _GROUPS_PER_CHUNK)], idx_v)
        copies = []
        for j in range(_GROUPS_PER_CHUNK):
            copies.append(
                pltpu.async_copy(
                    table_hbm.at[idx_v.at[j]],
                    rows_v.at[pl.ds(j * _GROUP, _GROUP)],
                    sem,
                )
            )
        for cp in copies:
            cp.wait()
        pltpu.sync_copy(rows_v, out_hbm.at[pl.ds(g0 * _GROUP, _CHUNK)])
        return carry

    lax.fori_loop(0, _CHUNKS_PER_W, chunk_step, 0)


@functools.lru_cache(maxsize=None)
def _make_sc_gather():
    @functools.partial(
        pl.kernel,
        out_type=jax.ShapeDtypeStruct((_BF, _DP), jnp.float32),
        mesh=plsc.VectorSubcoreMesh(core_axis_name="c", subcore_axis_name="s"),
        scratch_types=[
            pltpu.VMEM((_GROUPS_PER_CHUNK, _GROUP), jnp.int32),
            pltpu.VMEM((_CHUNK, _DP), jnp.float32),
            pltpu.SemaphoreType.DMA,
        ],
        compiler_params=pltpu.CompilerParams(use_tc_tiling_on_sc=False),
    )
    def _sc_gather(table_hbm, idx_hbm, out_hbm, idx_v, rows_v, sem):
        _sc_gather_body(table_hbm, idx_hbm, out_hbm, idx_v, rows_v, sem)

    return _sc_gather


_MLP_BLK = 1024
_EMBW = _F * _DP


def _mlp_body(xn_ref, emb_ref, w1a_ref, w1b_ref, b1_ref, w2_ref, b2_ref,
              w3_ref, b3_ref, out_ref):
    h = jnp.dot(xn_ref[...], w1a_ref[...], preferred_element_type=jnp.float32)
    h = h + jnp.dot(emb_ref[...], w1b_ref[...],
                    preferred_element_type=jnp.float32)
    h = jnp.maximum(h + b1_ref[...], 0.0)
    h = jnp.maximum(
        jnp.dot(h, w2_ref[...], preferred_element_type=jnp.float32)
        + b2_ref[...], 0.0)
    out_ref[...] = (
        jnp.dot(h, w3_ref[...], preferred_element_type=jnp.float32)
        + b3_ref[...])


def _tc_mlp(x_num, emb, W1a, W1b, b1, W2, b2, W3, b3):
    grid = (_B // _MLP_BLK,)
    return pl.pallas_call(
        _mlp_body,
        grid=grid,
        in_specs=[
            pl.BlockSpec((_MLP_BLK, _NUM), lambda i: (i, 0)),
            pl.BlockSpec((_MLP_BLK, _EMBW), lambda i: (i, 0)),
            pl.BlockSpec((_NUM, 512), lambda i: (0, 0)),
            pl.BlockSpec((_EMBW, 512), lambda i: (0, 0)),
            pl.BlockSpec((1, 512), lambda i: (0, 0)),
            pl.BlockSpec((512, 256), lambda i: (0, 0)),
            pl.BlockSpec((1, 256), lambda i: (0, 0)),
            pl.BlockSpec((256, 1), lambda i: (0, 0)),
            pl.BlockSpec((1, 1), lambda i: (0, 0)),
        ],
        out_specs=pl.BlockSpec((_MLP_BLK, 1), lambda i: (i, 0)),
        out_shape=jax.ShapeDtypeStruct((_B, 1), jnp.float32),
    )(x_num, emb, W1a, W1b, b1, W2, b2, W3, b3)


def kernel(x_num, x_cat, tables, W1, b1, W2, b2, W3, b3):
    table_pad = _tc_pad(tables)
    idx = (x_cat.astype(jnp.int32)
           + jnp.arange(_F, dtype=jnp.int32)[None, :] * _V)
    idx2 = idx.reshape(_BF // _GROUP, _GROUP)
    emb = _make_sc_gather()(table_pad, idx2)
    emb2 = emb.reshape(_B, _EMBW)
    w1e = W1[_NUM:].reshape(_F, _D, 512)
    w1e = jnp.pad(w1e, ((0, 0), (0, _DP - _D), (0, 0)))
    w1e = w1e.reshape(_EMBW, 512)
    out = _tc_mlp(
        x_num, emb2,
        W1[:_NUM], w1e,
        b1.reshape(1, 512), W2, b2.reshape(1, 256), W3, b3.reshape(1, 1))
    return out


# speedup vs baseline: 3.6431x; 1.8615x over previous
import functools

import jax
import jax.numpy as jnp
from jax import lax
from jax.experimental import pallas as pl
from jax.experimental.pallas import tpu as pltpu
from jax.experimental.pallas import tpu_sc as plsc

_B = 16384
_F = 26
_V = 100000
_D = 50
_DP = 128
_NUM = 13
_BF = _B * _F

_NC = 2
_NS = 16
_NW = _NC * _NS

_ROWS_PER_W = _BF // _NW
_GROUP = 128
_GROUPS_PER_CHUNK = 4
_CHUNK = _GROUP * _GROUPS_PER_CHUNK
_CHUNKS_PER_W = _ROWS_PER_W // _CHUNK

_VCHUNKS = [(k * 6400, 6400) for k in range(15)] + [(96000, 4000)]
_VCMAX = 6400


def _in_copy(t_ref, xin_v, in_sems, f, b):
    return pltpu.make_async_copy(t_ref.at[f], xin_v.at[b], in_sems.at[b])


def _pad_body(t_ref, out_ref, xin_v, xt_v, in_sems, out_sems):
    f = pl.program_id(0)
    nf = pl.num_programs(0)

    @pl.when(f == 0)
    def _():
        _in_copy(t_ref, xin_v, in_sems, 0, 0).start()

    @pl.when(f + 1 < nf)
    def _():
        _in_copy(t_ref, xin_v, in_sems, f + 1, (f + 1) % 2).start()

    _in_copy(t_ref, xin_v, in_sems, f, f % 2).wait()

    x = xin_v.at[f % 2]
    for j, (v0, vn) in enumerate(_VCHUNKS):
        xc = x[:, pl.ds(v0, vn)]
        xt = jnp.swapaxes(xc, 0, 1)
        z = jnp.zeros((vn, _DP - _D), dtype=jnp.float32)
        if j >= 2:
            v0p, vnp = _VCHUNKS[j - 2]
            pltpu.make_async_copy(
                xt_v.at[j % 2, pl.ds(0, vnp)],
                out_ref.at[pl.ds(f * _V + v0p, vnp)], out_sems.at[j % 2]).wait()
        xt_v[j % 2, pl.ds(0, vn)] = jnp.concatenate([xt, z], axis=1)
        pltpu.make_async_copy(
            xt_v.at[j % 2, pl.ds(0, vn)],
            out_ref.at[pl.ds(f * _V + v0, vn)], out_sems.at[j % 2]).start()
    for j in (len(_VCHUNKS) - 2, len(_VCHUNKS) - 1):
        v0, vn = _VCHUNKS[j]
        pltpu.make_async_copy(
            xt_v.at[j % 2, pl.ds(0, vn)],
            out_ref.at[pl.ds(f * _V + v0, vn)], out_sems.at[j % 2]).wait()


def _tc_pad(tables_t):
    return pl.pallas_call(
        _pad_body,
        grid=(_F,),
        in_specs=[pl.BlockSpec(memory_space=pltpu.HBM)],
        out_specs=pl.BlockSpec(memory_space=pltpu.HBM),
        out_shape=jax.ShapeDtypeStruct((_F * _V, _DP), jnp.float32),
        scratch_shapes=[
            pltpu.VMEM((2, _D, _V), jnp.float32),
            pltpu.VMEM((2, _VCMAX, _DP), jnp.float32),
            pltpu.SemaphoreType.DMA((2,)),
            pltpu.SemaphoreType.DMA((2,)),
        ],
    )(tables_t)


def _sc_gather_body(table_hbm, idx_hbm, out_hbm, idx_v, rows_v, sem):
    wid = lax.axis_index("s") * _NC + lax.axis_index("c")
    group_base = wid * (_ROWS_PER_W // _GROUP)

    def chunk_step(c, carry):
        g0 = group_base + c * _GROUPS_PER_CHUNK
        pltpu.sync_copy(idx_hbm.at[pl.ds(g0, _GROUPS_PER_CHUNK)], idx_v)
        copies = []
        for j in range(_GROUPS_PER_CHUNK):
            copies.append(
                pltpu.async_copy(
                    table_hbm.at[idx_v.at[j]],
                    rows_v.at[pl.ds(j * _GROUP, _GROUP)],
                    sem,
                )
            )
        for cp in copies:
            cp.wait()
        pltpu.sync_copy(rows_v, out_hbm.at[pl.ds(g0 * _GROUP, _CHUNK)])
        return carry

    lax.fori_loop(0, _CHUNKS_PER_W, chunk_step, 0)


@functools.lru_cache(maxsize=None)
def _make_sc_gather():
    @functools.partial(
        pl.kernel,
        out_type=jax.ShapeDtypeStruct((_BF, _DP), jnp.float32),
        mesh=plsc.VectorSubcoreMesh(core_axis_name="c", subcore_axis_name="s"),
        scratch_types=[
            pltpu.VMEM((_GROUPS_PER_CHUNK, _GROUP), jnp.int32),
            pltpu.VMEM((_CHUNK, _DP), jnp.float32),
            pltpu.SemaphoreType.DMA,
        ],
        compiler_params=pltpu.CompilerParams(use_tc_tiling_on_sc=False),
    )
    def _sc_gather(table_hbm, idx_hbm, out_hbm, idx_v, rows_v, sem):
        _sc_gather_body(table_hbm, idx_hbm, out_hbm, idx_v, rows_v, sem)

    return _sc_gather


_MLP_BLK = 1024
_EMBW = _F * _DP


def _mlp_body(xn_ref, emb_ref, w1a_ref, w1b_ref, b1_ref, w2_ref, b2_ref,
              w3_ref, b3_ref, out_ref):
    h = jnp.dot(xn_ref[...], w1a_ref[...], preferred_element_type=jnp.float32)
    h = h + jnp.dot(emb_ref[...], w1b_ref[...],
                    preferred_element_type=jnp.float32)
    h = jnp.maximum(h + b1_ref[...], 0.0)
    h = jnp.maximum(
        jnp.dot(h, w2_ref[...], preferred_element_type=jnp.float32)
        + b2_ref[...], 0.0)
    out_ref[...] = (
        jnp.dot(h, w3_ref[...], preferred_element_type=jnp.float32)
        + b3_ref[...])


def _tc_mlp(x_num, emb, W1a, W1b, b1, W2, b2, W3, b3):
    grid = (_B // _MLP_BLK,)
    return pl.pallas_call(
        _mlp_body,
        grid=grid,
        in_specs=[
            pl.BlockSpec((_MLP_BLK, _NUM), lambda i: (i, 0)),
            pl.BlockSpec((_MLP_BLK, _EMBW), lambda i: (i, 0)),
            pl.BlockSpec((_NUM, 512), lambda i: (0, 0)),
            pl.BlockSpec((_EMBW, 512), lambda i: (0, 0)),
            pl.BlockSpec((1, 512), lambda i: (0, 0)),
            pl.BlockSpec((512, 256), lambda i: (0, 0)),
            pl.BlockSpec((1, 256), lambda i: (0, 0)),
            pl.BlockSpec((256, 1), lambda i: (0, 0)),
            pl.BlockSpec((1, 1), lambda i: (0, 0)),
        ],
        out_specs=pl.BlockSpec((_MLP_BLK, 1), lambda i: (i, 0)),
        out_shape=jax.ShapeDtypeStruct((_B, 1), jnp.float32),
    )(x_num, emb, W1a, W1b, b1, W2, b2, W3, b3)


def kernel(x_num, x_cat, tables, W1, b1, W2, b2, W3, b3):
    table_pad = _tc_pad(jnp.transpose(tables, (0, 2, 1)))
    idx = (x_cat.astype(jnp.int32)
           + jnp.arange(_F, dtype=jnp.int32)[None, :] * _V)
    idx2 = idx.reshape(_BF // _GROUP, _GROUP)
    emb = _make_sc_gather()(table_pad, idx2)
    emb2 = emb.reshape(_B, _EMBW)
    w1e = W1[_NUM:].reshape(_F, _D, 512)
    w1e = jnp.pad(w1e, ((0, 0), (0, _DP - _D), (0, 0)))
    w1e = w1e.reshape(_EMBW, 512)
    out = _tc_mlp(
        x_num, emb2,
        W1[:_NUM], w1e,
        b1.reshape(1, 512), W2, b2.reshape(1, 256), W3, b3.reshape(1, 1))
    return out


# speedup vs baseline: 4.2180x; 1.1578x over previous
import functools

import jax
import jax.numpy as jnp
from jax import lax
from jax.experimental import pallas as pl
from jax.experimental.pallas import tpu as pltpu
from jax.experimental.pallas import tpu_sc as plsc

_B = 16384
_F = 26
_V = 100000
_D = 50
_DP = 128
_NUM = 13
_BF = _B * _F

_NC = 2
_NS = 16
_NW = _NC * _NS

_ROWS_PER_W = _BF // _NW
_GROUP = 128
_GROUPS_PER_CHUNK = 4
_CHUNK = _GROUP * _GROUPS_PER_CHUNK
_CHUNKS_PER_W = _ROWS_PER_W // _CHUNK

_VCHUNKS = [(k * 6400, 6400) for k in range(15)] + [(96000, 4000)]
_VCMAX = 6400


def _in_copy(t_ref, xin_v, in_sems, f, b):
    return pltpu.make_async_copy(t_ref.at[f], xin_v.at[b], in_sems.at[b])


def _pad_body(t_ref, out_ref, xin_v, xt_v, in_sems, out_sems):
    f = pl.program_id(0)
    nf = pl.num_programs(0)

    @pl.when(f == 0)
    def _():
        _in_copy(t_ref, xin_v, in_sems, 0, 0).start()

    @pl.when(f + 1 < nf)
    def _():
        _in_copy(t_ref, xin_v, in_sems, f + 1, (f + 1) % 2).start()

    _in_copy(t_ref, xin_v, in_sems, f, f % 2).wait()

    x = xin_v.at[f % 2]
    for j, (v0, vn) in enumerate(_VCHUNKS):
        xc = x[:, pl.ds(v0, vn)]
        xt = jnp.swapaxes(xc, 0, 1)
        z = jnp.zeros((vn, _DP - _D), dtype=jnp.float32)
        if j >= 2:
            v0p, vnp = _VCHUNKS[j - 2]
            pltpu.make_async_copy(
                xt_v.at[j % 2, pl.ds(0, vnp)],
                out_ref.at[pl.ds(f * _V + v0p, vnp)], out_sems.at[j % 2]).wait()
        xt_v[j % 2, pl.ds(0, vn)] = jnp.concatenate([xt, z], axis=1)
        pltpu.make_async_copy(
            xt_v.at[j % 2, pl.ds(0, vn)],
            out_ref.at[pl.ds(f * _V + v0, vn)], out_sems.at[j % 2]).start()
    for j in (len(_VCHUNKS) - 2, len(_VCHUNKS) - 1):
        v0, vn = _VCHUNKS[j]
        pltpu.make_async_copy(
            xt_v.at[j % 2, pl.ds(0, vn)],
            out_ref.at[pl.ds(f * _V + v0, vn)], out_sems.at[j % 2]).wait()


def _tc_pad(tables_t):
    return pl.pallas_call(
        _pad_body,
        grid=(_F,),
        in_specs=[pl.BlockSpec(memory_space=pltpu.HBM)],
        out_specs=pl.BlockSpec(memory_space=pltpu.HBM),
        out_shape=jax.ShapeDtypeStruct((_F * _V, _DP), jnp.float32),
        scratch_shapes=[
            pltpu.VMEM((2, _D, _V), jnp.float32),
            pltpu.VMEM((2, _VCMAX, _DP), jnp.float32),
            pltpu.SemaphoreType.DMA((2,)),
            pltpu.SemaphoreType.DMA((2,)),
        ],
    )(tables_t)


def _sc_gather_body(table_hbm, idx_hbm, out_hbm, idx_v, rows_v, sem):
    wid = lax.axis_index("s") * _NC + lax.axis_index("c")
    group_base = wid * (_ROWS_PER_W // _GROUP)

    def chunk_step(c, carry):
        g0 = group_base + c * _GROUPS_PER_CHUNK
        pltpu.sync_copy(idx_hbm.at[pl.ds(g0, _GROUPS_PER_CHUNK)], idx_v)
        copies = []
        for j in range(_GROUPS_PER_CHUNK):
            copies.append(
                pltpu.async_copy(
                    table_hbm.at[idx_v.at[j]],
                    rows_v.at[pl.ds(j * _GROUP, _GROUP)],
                    sem,
                )
            )
        for cp in copies:
            cp.wait()
        pltpu.sync_copy(rows_v, out_hbm.at[pl.ds(g0 * _GROUP, _CHUNK)])
        return carry

    lax.fori_loop(0, _CHUNKS_PER_W, chunk_step, 0)


@functools.lru_cache(maxsize=None)
def _make_sc_gather():
    @functools.partial(
        pl.kernel,
        out_type=jax.ShapeDtypeStruct((_BF, _DP), jnp.float32),
        mesh=plsc.VectorSubcoreMesh(core_axis_name="c", subcore_axis_name="s"),
        scratch_types=[
            pltpu.VMEM((_GROUPS_PER_CHUNK, _GROUP), jnp.int32),
            pltpu.VMEM((_CHUNK, _DP), jnp.float32),
            pltpu.SemaphoreType.DMA,
        ],
        compiler_params=pltpu.CompilerParams(use_tc_tiling_on_sc=False),
    )
    def _sc_gather(table_hbm, idx_hbm, out_hbm, idx_v, rows_v, sem):
        _sc_gather_body(table_hbm, idx_hbm, out_hbm, idx_v, rows_v, sem)

    return _sc_gather


_MLP_BLK = 1024
_EMBW = _F * _DP


def _mlp_body(xn_ref, emb_ref, w1a_ref, w1b_ref, b1_ref, w2_ref, b2_ref,
              w3_ref, b3_ref, out_ref):
    e = emb_ref[...].reshape(_MLP_BLK, _EMBW)
    h = jnp.dot(xn_ref[...], w1a_ref[...], preferred_element_type=jnp.float32)
    h = h + jnp.dot(e, w1b_ref[...],
                    preferred_element_type=jnp.float32)
    h = jnp.maximum(h + b1_ref[...], 0.0)
    h = jnp.maximum(
        jnp.dot(h, w2_ref[...], preferred_element_type=jnp.float32)
        + b2_ref[...], 0.0)
    out_ref[...] = (
        jnp.dot(h, w3_ref[...], preferred_element_type=jnp.float32)
        + b3_ref[...])


def _tc_mlp(x_num, emb, W1a, W1b, b1, W2, b2, W3, b3):
    grid = (_B // _MLP_BLK,)
    return pl.pallas_call(
        _mlp_body,
        grid=grid,
        in_specs=[
            pl.BlockSpec((_MLP_BLK, _NUM), lambda i: (i, 0)),
            pl.BlockSpec((_MLP_BLK * _F, _DP), lambda i: (i, 0)),
            pl.BlockSpec((_NUM, 512), lambda i: (0, 0)),
            pl.BlockSpec((_EMBW, 512), lambda i: (0, 0)),
            pl.BlockSpec((1, 512), lambda i: (0, 0)),
            pl.BlockSpec((512, 256), lambda i: (0, 0)),
            pl.BlockSpec((1, 256), lambda i: (0, 0)),
            pl.BlockSpec((256, 1), lambda i: (0, 0)),
            pl.BlockSpec((1, 1), lambda i: (0, 0)),
        ],
        out_specs=pl.BlockSpec((_MLP_BLK, 1), lambda i: (i, 0)),
        out_shape=jax.ShapeDtypeStruct((_B, 1), jnp.float32),
    )(x_num, emb, W1a, W1b, b1, W2, b2, W3, b3)


def kernel(x_num, x_cat, tables, W1, b1, W2, b2, W3, b3):
    table_pad = _tc_pad(jnp.transpose(tables, (0, 2, 1)))
    idx = (x_cat.astype(jnp.int32)
           + jnp.arange(_F, dtype=jnp.int32)[None, :] * _V)
    idx2 = idx.reshape(_BF // _GROUP, _GROUP)
    emb = _make_sc_gather()(table_pad, idx2)
    w1e = W1[_NUM:].reshape(_F, _D, 512)
    w1e = jnp.pad(w1e, ((0, 0), (0, _DP - _D), (0, 0)))
    w1e = w1e.reshape(_EMBW, 512)
    out = _tc_mlp(
        x_num, emb,
        W1[:_NUM], w1e,
        b1.reshape(1, 512), W2, b2.reshape(1, 256), W3, b3.reshape(1, 1))
    return out
